# baseline (speedup 1.0000x reference)
import functools

import jax
import jax.numpy as jnp
from jax import lax
from jax.experimental import pallas as pl
from jax.experimental.pallas import tpu as pltpu
from jax.experimental.pallas import tpu_sc as plsc

N = 10000
D = 128
H = 64
E = 320000

NC = 2
NS = 16
NW = NC * NS
GC = 125
NCHG = E // GC
NCH = NCHG // NW
DC = 1000
NCHD = E // DC // NS
ZCH = 1000
ZR = GC
BLK = 2000

_mesh = plsc.VectorSubcoreMesh(core_axis_name="c", subcore_axis_name="s")
_sc_params = pltpu.CompilerParams(use_tc_tiling_on_sc=False)


@functools.partial(
    pl.kernel,
    out_type=jax.ShapeDtypeStruct((NC, 1, N), jnp.float32),
    mesh=_mesh,
    compiler_params=_sc_params,
    scratch_types=[
        pltpu.VMEM((NCHD, DC), jnp.int32),
        pltpu.VMEM((DC,), jnp.float32),
        pltpu.VMEM((ZCH,), jnp.float32),
        pltpu.VMEM((N,), jnp.float32),
        pltpu.VMEM_SHARED((N,), jnp.float32),
        pltpu.SemaphoreType.DMA,
        pltpu.SemaphoreType.DMA,
    ],
)
def _deg_kernel(dst2_hbm, ones_hbm, zeros_hbm, out_hbm,
                idx_v, ones_v, zb, wb, acc, sem0, sem1):
    c = lax.axis_index("c")
    s = lax.axis_index("s")

    pltpu.sync_copy(zeros_hbm, zb)
    for j in range(N // ZCH):
        @pl.when(s == (j % NS))
        def _(j=j):
            pltpu.sync_copy(zb, acc.at[pl.ds(j * ZCH, ZCH)])

    pltpu.sync_copy(ones_hbm, ones_v)
    plsc.subcore_barrier()

    pltpu.sync_copy(dst2_hbm.at[c, pl.ds(s * NCHD, NCHD)], idx_v)
    sems = (sem0, sem1)

    def body(p, carry):
        for b in range(2):
            j = 2 * p + b

            @pl.when(j >= 2)
            def _(b=b):
                pltpu.make_async_copy(ones_hbm, ones_v, sems[b]).wait()

            pltpu.async_copy(ones_v, acc.at[idx_v.at[j]], sems[b], add=True)
        return carry

    lax.fori_loop(0, NCHD // 2, body, 0)
    pltpu.make_async_copy(ones_hbm, ones_v, sem0).wait()
    pltpu.make_async_copy(ones_hbm, ones_v, sem1).wait()
    plsc.subcore_barrier()

    @pl.when(s == 0)
    def _():
        pltpu.sync_copy(acc, wb)
        pltpu.sync_copy(wb, out_hbm.at[c, 0])


@functools.partial(
    pl.kernel,
    out_type=jax.ShapeDtypeStruct((NC, N, H), jnp.float32),
    mesh=_mesh,
    compiler_params=_sc_params,
    scratch_types=[
        pltpu.VMEM((NCH, GC), jnp.int32),
        pltpu.VMEM((NCH, GC), jnp.int32),
        pltpu.VMEM((GC, H), jnp.float32),
        pltpu.VMEM((GC, H), jnp.float32),
        pltpu.VMEM((GC, H), jnp.float32),
        pltpu.VMEM((GC, H), jnp.float32),
        pltpu.VMEM_SHARED((N, H), jnp.float32),
        pltpu.SemaphoreType.DMA,
        pltpu.SemaphoreType.DMA,
        pltpu.SemaphoreType.DMA,
        pltpu.SemaphoreType.DMA,
        pltpu.SemaphoreType.DMA,
        pltpu.SemaphoreType.DMA,
        pltpu.SemaphoreType.DMA,
        pltpu.SemaphoreType.DMA,
    ],
)
def _scatter_kernel(tab_hbm, ei_hbm, zrows_hbm, out_hbm,
                    idx_s, idx_d, rows0, rows1, rows2, rows3, acc,
                    gsem0, gsem1, gsem2, gsem3, ssem0, ssem1, ssem2, ssem3):
    c = lax.axis_index("c")
    s = lax.axis_index("s")
    wid = s * NC + c
    rows = (rows0, rows1, rows2, rows3)
    gsem = (gsem0, gsem1, gsem2, gsem3)
    ssem = (ssem0, ssem1, ssem2, ssem3)

    pltpu.async_copy(ei_hbm.at[0, pl.ds(wid * NCH, NCH)], idx_s, gsem0)
    pltpu.async_copy(ei_hbm.at[1, pl.ds(wid * NCH, NCH)], idx_d, gsem1)

    NZ = N // ZR // NS
    pltpu.sync_copy(zrows_hbm, rows0)
    for k in range(NZ):
        pltpu.async_copy(rows0, acc.at[pl.ds((NZ * s + k) * ZR, ZR)],
                         ssem[k % 4])
    for k in range(min(NZ, 4)):
        pltpu.make_async_copy(zrows_hbm, rows0, ssem[k]).wait()
    if NZ > 4:
        for k in range(4, NZ):
            pltpu.make_async_copy(zrows_hbm, rows0, ssem[k % 4]).wait()
    pltpu.make_async_copy(ei_hbm.at[0, pl.ds(0, NCH)], idx_s, gsem0).wait()
    pltpu.make_async_copy(ei_hbm.at[0, pl.ds(0, NCH)], idx_d, gsem1).wait()

    plsc.subcore_barrier()

    for b in range(3):
        pltpu.async_copy(tab_hbm.at[idx_s.at[b]], rows[b], gsem[b])

    def ring(p, carry):
        for b in range(4):
            t = 4 * p + b
            a = (b + 3) % 4

            @pl.when(t > 0)
            def _(a=a):
                pltpu.make_async_copy(zrows_hbm, rows[a], ssem[a]).wait()

            @pl.when(t + 3 < NCH)
            def _(t=t, a=a):
                pltpu.async_copy(tab_hbm.at[idx_s.at[t + 3]], rows[a],
                                 gsem[a])

            pltpu.make_async_copy(zrows_hbm, rows[b], gsem[b]).wait()
            pltpu.async_copy(rows[b], acc.at[idx_d.at[t]], ssem[b],
                             add=True)
        return carry

    lax.fori_loop(0, NCH // 4, ring, 0)
    pltpu.make_async_copy(zrows_hbm, rows[(NCH - 1) % 4], ssem[(NCH - 1) % 4]).wait()

    plsc.subcore_barrier()
    NZ2 = N // ZR // NS
    for k in range(NZ2):
        b = k % 4
        if k >= 4:
            pltpu.make_async_copy(zrows_hbm, rows[b], ssem[b]).wait()
        j = NZ2 * s + k
        pltpu.async_copy(acc.at[pl.ds(j * ZR, ZR)], rows[b], gsem[b])
        pltpu.make_async_copy(zrows_hbm, rows[b], gsem[b]).wait()
        pltpu.async_copy(rows[b], out_hbm.at[c, pl.ds(j * ZR, ZR)], ssem[b])
    for k in range(max(NZ2 - 4, 0), NZ2):
        pltpu.make_async_copy(zrows_hbm, rows[k % 4], ssem[k % 4]).wait()


N2 = N // 2
H2 = 2 * H
BLK2 = 1000


def _scale(dp):
    dinv = lax.rsqrt(dp + 1.0)
    return jnp.concatenate(
        [jnp.broadcast_to(dinv[:, 0:1], (dp.shape[0], H)),
         jnp.broadcast_to(dinv[:, 1:2], (dp.shape[0], H))], axis=1)


def _pair_dot(x, w, b):
    he = jnp.dot(x[:, :x.shape[1] // 2], w,
                 preferred_element_type=jnp.float32) + b
    ho = jnp.dot(x[:, x.shape[1] // 2:], w,
                 preferred_element_type=jnp.float32) + b
    return jnp.concatenate([he, ho], axis=1)


def _tc_a_body(x_ref, w_ref, b_ref, dp_ref, h_ref, g_ref):
    h = _pair_dot(x_ref[...], w_ref[...], b_ref[...])
    sc = _scale(dp_ref[...])
    h_ref[...] = h
    g_ref[...] = sc * h


_tc_a = pl.pallas_call(
    _tc_a_body,
    grid=(N2 // BLK2,),
    in_specs=[
        pl.BlockSpec((BLK2, 2 * D), lambda i: (i, 0)),
        pl.BlockSpec((D, H), lambda i: (0, 0)),
        pl.BlockSpec((1, H), lambda i: (0, 0)),
        pl.BlockSpec((BLK2, 2), lambda i: (i, 0)),
    ],
    out_specs=[pl.BlockSpec((BLK2, H2), lambda i: (i, 0))] * 2,
    out_shape=[jax.ShapeDtypeStruct((N2, H2), jnp.float32)] * 2,
)


def _tc_b_body(sp_ref, h1_ref, dp_ref, w_ref, b_ref, h2_ref, g2_ref):
    sc = _scale(dp_ref[...])
    ssum = sp_ref[0] + sp_ref[1]
    z = jnp.maximum(sc * ssum + (sc * sc) * h1_ref[...], 0.0)
    h2 = _pair_dot(z, w_ref[...], b_ref[...])
    h2_ref[...] = h2
    g2_ref[...] = sc * h2


_tc_b = pl.pallas_call(
    _tc_b_body,
    grid=(N2 // BLK2,),
    in_specs=[
        pl.BlockSpec((NC, BLK2, H2), lambda i: (0, i, 0)),
        pl.BlockSpec((BLK2, H2), lambda i: (i, 0)),
        pl.BlockSpec((BLK2, 2), lambda i: (i, 0)),
        pl.BlockSpec((H, H), lambda i: (0, 0)),
        pl.BlockSpec((1, H), lambda i: (0, 0)),
    ],
    out_specs=[pl.BlockSpec((BLK2, H2), lambda i: (i, 0))] * 2,
    out_shape=[jax.ShapeDtypeStruct((N2, H2), jnp.float32)] * 2,
)


def _tc_c_body(sp_ref, h2_ref, dp_ref, o_ref):
    sc = _scale(dp_ref[...])
    ssum = sp_ref[0] + sp_ref[1]
    o_ref[...] = sc * ssum + (sc * sc) * h2_ref[...]


_tc_c = pl.pallas_call(
    _tc_c_body,
    grid=(N2 // BLK2,),
    in_specs=[
        pl.BlockSpec((NC, BLK2, H2), lambda i: (0, i, 0)),
        pl.BlockSpec((BLK2, H2), lambda i: (i, 0)),
        pl.BlockSpec((BLK2, 2), lambda i: (i, 0)),
    ],
    out_specs=pl.BlockSpec((BLK2, H2), lambda i: (i, 0)),
    out_shape=jax.ShapeDtypeStruct((N2, H2), jnp.float32),
)


def kernel(features_plus, features_minus, edge_index_pos, edge_index_neg,
           Wp1, bp1, Wp2, bp2, Wn1, bn1, Wn2, bn2):
    eip = edge_index_pos.reshape(2, NCHG, GC)
    ein = edge_index_neg.reshape(2, NCHG, GC)
    ones = jnp.ones((DC,), jnp.float32)
    zeros1 = jnp.zeros((ZCH,), jnp.float32)
    zrows = jnp.zeros((ZR, H), jnp.float32)

    dst2 = jnp.stack([edge_index_pos[1], edge_index_neg[1]]
                     ).reshape(2, E // DC, DC)
    degb = _deg_kernel(dst2, ones, zeros1)
    dpp = degb[0].reshape(N2, 2)
    dpn = degb[1].reshape(N2, 2)

    xp2 = features_plus.reshape(N2, 2 * D)
    xn2 = features_minus.reshape(N2, 2 * D)

    h1p, g1p = _tc_a(xp2, Wp1, bp1.reshape(1, H), dpp)
    h1n, g1n = _tc_a(xn2, Wn1, bn1.reshape(1, H), dpn)

    s1p = _scatter_kernel(g1p.reshape(N, H), eip, zrows)
    s1n = _scatter_kernel(g1n.reshape(N, H), ein, zrows)

    h2p, g2p = _tc_b(s1p.reshape(NC, N2, H2), h1p, dpp, Wp2, bp2.reshape(1, H))
    h2n, g2n = _tc_b(s1n.reshape(NC, N2, H2), h1n, dpn, Wn2, bn2.reshape(1, H))

    s2p = _scatter_kernel(g2p.reshape(N, H), eip, zrows)
    s2n = _scatter_kernel(g2n.reshape(N, H), ein, zrows)

    x = _tc_c(s2p.reshape(NC, N2, H2), h2p, dpp)
    y = _tc_c(s2n.reshape(NC, N2, H2), h2n, dpn)
    return (x.reshape(N, H), y.reshape(N, H))
